# baseline (speedup 1.0000x reference)
import functools

import jax
import jax.numpy as jnp
from jax import lax
from jax.experimental import pallas as pl
from jax.experimental.pallas import tpu as pltpu
from jax.experimental.pallas import tpu_sc as plsc

N = 10000
D = 128
DW = 144
E = 320000
NC = 2
NS = 16
NW = NC * NS
PN = 10112
EPW = 10240
ROWS_PER_TILE = PN // NS
CH = 128
STEPS = EPW // CH
STG = 8
NPAD = 16


def _sc_body(xu, xi, sF, dF, sR, dR, sS, dS, accp,
             idxs, idxd, rows0, rows1, gsem0, gsem1, ssem0, ssem1, zsem,
             acc):
    rows = (rows0, rows1)
    gsems = (gsem0, gsem1)
    ssems = (ssem0, ssem1)
    c = lax.axis_index("c")
    s = lax.axis_index("s")
    wid = s * NC + c
    row0 = s * ROWS_PER_TILE

    _zchunks = tuple((k * 112, min(112, ROWS_PER_TILE - k * 112))
                     for k in range((ROWS_PER_TILE + 111) // 112))
    for k, nr in _zchunks:
        pltpu.async_copy(xu.at[pl.ds(N, nr)], acc.at[pl.ds(row0 + k, nr)],
                         zsem)
    for k, nr in _zchunks:
        pltpu.make_async_copy(xu.at[pl.ds(N, nr)],
                              acc.at[pl.ds(row0 + k, nr)], zsem).wait()
    plsc.subcore_barrier()

    for rel, (x_hbm, src_hbm, dst_hbm) in enumerate(
        ((xu, sF, dF), (xu, sR, dR), (xi, sS, dS))
    ):
        def _gather(i, b):
            pltpu.async_copy(x_hbm.at[idxs.at[i]], rows[b], gsems[b])

        def _gwait(i, b):
            pltpu.make_async_copy(x_hbm.at[idxs.at[i]], rows[b],
                                  gsems[b]).wait()

        def _scat(i, b):
            pltpu.async_copy(rows[b], acc.at[idxd.at[i]], ssems[b], add=True)

        def _swait(b):
            pltpu.make_async_copy(rows[b], acc.at[idxd.at[0]],
                                  ssems[b]).wait()

        for h in range(STEPS // STG):
            pltpu.sync_copy(src_hbm.at[wid, pl.ds(h * STG, STG)], idxs)
            pltpu.sync_copy(dst_hbm.at[wid, pl.ds(h * STG, STG)], idxd)

            _gather(0, 0)
            _gwait(0, 0)
            _scat(0, 0)
            _gather(1, 1)

            def _pair(k, carry):
                for j in range(2):
                    i = 1 + 2 * k + j
                    b = (1 + j) % 2
                    _gwait(i, b)
                    _scat(i, b)
                    _swait(1 - b)
                    _gather(i + 1, 1 - b)
                return carry

            lax.fori_loop(0, (STG - 2) // 2, _pair, 0)
            _gwait(STG - 1, (STG - 1) % 2)
            _scat(STG - 1, (STG - 1) % 2)
            _swait(0)
            _swait(1)

        plsc.subcore_barrier()
        pltpu.sync_copy(acc.at[pl.ds(row0, ROWS_PER_TILE)],
                        accp.at[rel, c, pl.ds(row0, ROWS_PER_TILE)])
        plsc.subcore_barrier()


_sc_scatter = functools.partial(
    pl.kernel,
    out_type=jax.ShapeDtypeStruct((3, NC, PN, DW), jnp.float32),
    mesh=plsc.VectorSubcoreMesh(core_axis_name="c", subcore_axis_name="s"),
    compiler_params=pltpu.CompilerParams(use_tc_tiling_on_sc=False),
    scratch_types=[
        pltpu.VMEM((STG, CH), jnp.int32),
        pltpu.VMEM((STG, CH), jnp.int32),
        pltpu.VMEM((CH, DW), jnp.float32),
        pltpu.VMEM((CH, DW), jnp.float32),
        pltpu.SemaphoreType.DMA,
        pltpu.SemaphoreType.DMA,
        pltpu.SemaphoreType.DMA,
        pltpu.SemaphoreType.DMA,
        pltpu.SemaphoreType.DMA,
        pltpu.VMEM_SHARED((PN, DW), jnp.float32),
    ],
)(_sc_body)


def _tc_body(accp, wf, wr, ws, ou, oi):
    p0 = accp[0, 0] + accp[0, 1]
    p1 = accp[1, 0] + accp[1, 1]
    p2 = accp[2, 0] + accp[2, 1]
    s1 = p1 - p0
    s2 = p2 - p1

    def norm(sarr):
        a = sarr[:, :D]
        dg = jnp.maximum(sarr[:, D:D + 1], 1.0)
        return a / dg

    ou[...] = jnp.dot(norm(p0), wf[...], preferred_element_type=jnp.float32)
    oi[...] = (jnp.dot(norm(s1), wr[...], preferred_element_type=jnp.float32)
               + jnp.dot(norm(s2), ws[...], preferred_element_type=jnp.float32))


_TB = 632

_tc_combine = pl.pallas_call(
    _tc_body,
    grid=(PN // _TB,),
    in_specs=[
        pl.BlockSpec((3, NC, _TB, DW), lambda i: (0, 0, i, 0)),
        pl.BlockSpec((D, D), lambda i: (0, 0)),
        pl.BlockSpec((D, D), lambda i: (0, 0)),
        pl.BlockSpec((D, D), lambda i: (0, 0)),
    ],
    out_specs=[
        pl.BlockSpec((_TB, D), lambda i: (i, 0)),
        pl.BlockSpec((_TB, D), lambda i: (i, 0)),
    ],
    out_shape=[
        jax.ShapeDtypeStruct((PN, D), jnp.float32),
        jax.ShapeDtypeStruct((PN, D), jnp.float32),
    ],
)


def _prep_edges(e):
    pad = N + (jnp.arange(EPW - E // NW, dtype=jnp.int32) % NPAD)
    pad = jnp.broadcast_to(pad, (NW, EPW - E // NW))

    def one(v):
        v = v.reshape(NW, E // NW)
        v = jnp.concatenate([v, pad], axis=1)
        return v.reshape(NW, STEPS, CH)

    return one(e[0]), one(e[1])


def _prep_x(x):
    body = jnp.concatenate(
        [x, jnp.ones((N, 1), jnp.float32), jnp.zeros((N, DW - D - 1),
                                                     jnp.float32)], axis=1)
    return jnp.concatenate([body, jnp.zeros((PN - N, DW), jnp.float32)],
                           axis=0)


def kernel(x_user, x_item, edge_follows, edge_rates, edge_similar,
           W_follows, W_rates, W_similar):
    xu = _prep_x(x_user)
    xi = _prep_x(x_item)
    sF, dF = _prep_edges(edge_follows)
    sR, dR = _prep_edges(edge_rates)
    sS, dS = _prep_edges(edge_similar)

    accp = _sc_scatter(xu, xi, sF, dF, sR, dR, sS, dS)
    ou, oi = _tc_combine(accp, W_follows, W_rates, W_similar)
    return ou[:N], oi[:N]


# speedup vs baseline: 1.3402x; 1.3402x over previous
import functools

import jax
import jax.numpy as jnp
from jax import lax
from jax.experimental import pallas as pl
from jax.experimental.pallas import tpu as pltpu
from jax.experimental.pallas import tpu_sc as plsc

N = 10000
D = 128
E = 320000
NC = 2
NS = 16
NW = NC * NS
PN = 10240
EPW = 10240
ROWS_PER_TILE = PN // NS
CH = 64
STEPS = EPW // CH
QTR = STEPS // 4
NPAD = 16


def _sc_body(xu, xi, sF, dF, sR, dR, sS, dS, accp, degp,
             idxs, idxd, rows0, rows1, rows2, rows3, ones, zrows,
             gsem0, gsem1, gsem2, gsem3, ssem0, ssem1, ssem2, ssem3, dsem,
             acc, deg):
    rows = (rows0, rows1, rows2, rows3)
    gsems = (gsem0, gsem1, gsem2, gsem3)
    ssems = (ssem0, ssem1, ssem2, ssem3)
    c = lax.axis_index("c")
    s = lax.axis_index("s")
    wid = s * NC + c
    row0 = s * ROWS_PER_TILE

    zero16 = jnp.zeros((16,), jnp.float32)
    one16 = jnp.ones((16,), jnp.float32)
    for j in range(CH // 16):
        ones[pl.ds(j * 16, 16)] = one16

    def _zrow(i, carry):
        for j in range(8):
            zrows[i, pl.ds(j * 16, 16)] = zero16
        return carry

    lax.fori_loop(0, zrows.shape[0], _zrow, 0)

    ZR = zrows.shape[0]

    def _zacc(i, carry):
        pltpu.async_copy(zrows, acc.at[pl.ds(row0 + i * ZR, ZR)], dsem)
        return carry

    lax.fori_loop(0, ROWS_PER_TILE // ZR, _zacc, 0)

    def _zdeg(i, carry):
        pltpu.async_copy(zrows.at[0], deg.at[pl.ds(row0 + i * 128, 128)],
                         dsem)
        return carry

    lax.fori_loop(0, ROWS_PER_TILE // 128, _zdeg, 0)

    def _zdrain_a(i, carry):
        pltpu.make_async_copy(zrows, acc.at[pl.ds(row0, ZR)], dsem).wait()
        return carry

    lax.fori_loop(0, ROWS_PER_TILE // ZR, _zdrain_a, 0)

    def _zdrain_d(i, carry):
        pltpu.make_async_copy(zrows.at[0], deg.at[pl.ds(row0, 128)],
                              dsem).wait()
        return carry

    lax.fori_loop(0, ROWS_PER_TILE // 128, _zdrain_d, 0)
    plsc.subcore_barrier()

    for rel, (x_hbm, src_hbm, dst_hbm) in enumerate(
        ((xu, sF, dF), (xu, sR, dR), (xi, sS, dS))
    ):

        def _gather(i, b):
            pltpu.async_copy(x_hbm.at[idxs.at[i]], rows[b], gsems[b])

        def _gwait(i, b):
            pltpu.make_async_copy(x_hbm.at[idxs.at[i]], rows[b],
                                  gsems[b]).wait()

        def _swait(b):
            pltpu.make_async_copy(rows[b], acc.at[idxd.at[0]],
                                  ssems[b]).wait()

        def _process(i, b):
            _gwait(i, b)
            pltpu.async_copy(ones, deg.at[idxd.at[i]], dsem, add=True)
            pltpu.async_copy(rows[b], acc.at[idxd.at[i]], ssems[b], add=True)

        for h in range(4):
            pltpu.sync_copy(src_hbm.at[wid, pl.ds(h * QTR, QTR)], idxs)
            pltpu.sync_copy(dst_hbm.at[wid, pl.ds(h * QTR, QTR)], idxd)

            _gather(0, 0)
            _gather(1, 1)
            _process(0, 0)
            _gather(2, 2)
            _process(1, 1)
            _gather(3, 3)

            def _quad(k, carry):
                i0 = 2 + 4 * k
                for j in range(4):
                    i = i0 + j
                    b = (2 + j) % 4
                    _process(i, b)
                    bn = (b + 2) % 4
                    _swait(bn)
                    _gather(i + 2, bn)
                return carry

            lax.fori_loop(0, (QTR - 4) // 4, _quad, 0)
            _process(QTR - 2, (QTR - 2) % 4)
            _process(QTR - 1, (QTR - 1) % 4)
            for b in range(4):
                _swait(b)

            def _ddrain(i, carry):
                pltpu.make_async_copy(ones, deg.at[idxd.at[0]], dsem).wait()
                return carry

            lax.fori_loop(0, QTR, _ddrain, 0)
        plsc.subcore_barrier()

        pltpu.sync_copy(acc.at[pl.ds(row0, ROWS_PER_TILE)],
                        accp.at[rel, c, pl.ds(row0, ROWS_PER_TILE)])
        pltpu.sync_copy(deg.at[pl.ds(row0, ROWS_PER_TILE)],
                        degp.at[rel, c, pl.ds(row0, ROWS_PER_TILE)])
        plsc.subcore_barrier()


_sc_scatter = functools.partial(
    pl.kernel,
    out_type=(
        jax.ShapeDtypeStruct((3, NC, PN, D), jnp.float32),
        jax.ShapeDtypeStruct((3, NC, PN), jnp.float32),
    ),
    mesh=plsc.VectorSubcoreMesh(core_axis_name="c", subcore_axis_name="s"),
    scratch_types=[
        pltpu.VMEM((QTR, CH), jnp.int32),
        pltpu.VMEM((QTR, CH), jnp.int32),
        pltpu.VMEM((CH, D), jnp.float32),
        pltpu.VMEM((CH, D), jnp.float32),
        pltpu.VMEM((CH, D), jnp.float32),
        pltpu.VMEM((CH, D), jnp.float32),
        pltpu.VMEM((CH,), jnp.float32),
        pltpu.VMEM((4, D), jnp.float32),
        pltpu.SemaphoreType.DMA,
        pltpu.SemaphoreType.DMA,
        pltpu.SemaphoreType.DMA,
        pltpu.SemaphoreType.DMA,
        pltpu.SemaphoreType.DMA,
        pltpu.SemaphoreType.DMA,
        pltpu.SemaphoreType.DMA,
        pltpu.SemaphoreType.DMA,
        pltpu.SemaphoreType.DMA,
        pltpu.VMEM_SHARED((PN, D), jnp.float32),
        pltpu.VMEM_SHARED((PN,), jnp.float32),
    ],
)(_sc_body)


def _tc_body(accp, degp, wf, wr, ws, ou, oi):
    def norm(r):
        a = accp[r, 0] + accp[r, 1]
        dg = degp[r, 0] + degp[r, 1]
        if r > 0:
            a = a - (accp[r - 1, 0] + accp[r - 1, 1])
            dg = dg - (degp[r - 1, 0] + degp[r - 1, 1])
        return a / jnp.maximum(dg, 1.0)[:, None]

    ou[...] = jnp.dot(norm(0), wf[...], preferred_element_type=jnp.float32)
    oi[...] = (jnp.dot(norm(1), wr[...], preferred_element_type=jnp.float32)
               + jnp.dot(norm(2), ws[...], preferred_element_type=jnp.float32))


_TB = 1280

_tc_combine = pl.pallas_call(
    _tc_body,
    grid=(PN // _TB,),
    in_specs=[
        pl.BlockSpec((3, NC, _TB, D), lambda i: (0, 0, i, 0)),
        pl.BlockSpec((3, NC, _TB), lambda i: (0, 0, i)),
        pl.BlockSpec((D, D), lambda i: (0, 0)),
        pl.BlockSpec((D, D), lambda i: (0, 0)),
        pl.BlockSpec((D, D), lambda i: (0, 0)),
    ],
    out_specs=[
        pl.BlockSpec((_TB, D), lambda i: (i, 0)),
        pl.BlockSpec((_TB, D), lambda i: (i, 0)),
    ],
    out_shape=[
        jax.ShapeDtypeStruct((PN, D), jnp.float32),
        jax.ShapeDtypeStruct((PN, D), jnp.float32),
    ],
)


def _prep_edges(e):
    pad = N + (jnp.arange(EPW - E // NW, dtype=jnp.int32) % NPAD)
    pad = jnp.broadcast_to(pad, (NW, EPW - E // NW))

    def one(v):
        v = v.reshape(NW, E // NW)
        v = jnp.concatenate([v, pad], axis=1)
        return v.reshape(NW, STEPS, CH)

    return one(e[0]), one(e[1])


def kernel(x_user, x_item, edge_follows, edge_rates, edge_similar,
           W_follows, W_rates, W_similar):
    zpad = jnp.zeros((PN - N, D), jnp.float32)
    xu = jnp.concatenate([x_user, zpad], axis=0)
    xi = jnp.concatenate([x_item, zpad], axis=0)
    sF, dF = _prep_edges(edge_follows)
    sR, dR = _prep_edges(edge_rates)
    sS, dS = _prep_edges(edge_similar)

    accp, degp = _sc_scatter(xu, xi, sF, dF, sR, dR, sS, dS)
    ou, oi = _tc_combine(accp, degp, W_follows, W_rates, W_similar)
    return ou[:N], oi[:N]


# speedup vs baseline: 1.3434x; 1.0024x over previous
import functools

import jax
import jax.numpy as jnp
from jax import lax
from jax.experimental import pallas as pl
from jax.experimental.pallas import tpu as pltpu
from jax.experimental.pallas import tpu_sc as plsc

N = 10000
D = 128
E = 320000
NC = 2
NS = 16
NW = NC * NS
PN = 10240
EPW = 10240
ROWS_PER_TILE = PN // NS
CH = 64
STEPS = EPW // CH
QTR = STEPS // 10
NPAD = 16


def _sc_body(xu, xi, sF, dF, sR, dR, sS, dS, accp, degp,
             idxs0, idxd0, idxs1, idxd1, rows0, rows1, rows2, rows3, ones,
             zrows, gsem0, gsem1, gsem2, gsem3, ssem0, ssem1, ssem2, ssem3,
             dsem, isem, acc, deg):
    rows = (rows0, rows1, rows2, rows3)
    gsems = (gsem0, gsem1, gsem2, gsem3)
    ssems = (ssem0, ssem1, ssem2, ssem3)
    idxsets = ((idxs0, idxd0), (idxs1, idxd1))
    c = lax.axis_index("c")
    s = lax.axis_index("s")
    wid = s * NC + c
    row0 = s * ROWS_PER_TILE

    zero16 = jnp.zeros((16,), jnp.float32)
    one16 = jnp.ones((16,), jnp.float32)
    for j in range(CH // 16):
        ones[pl.ds(j * 16, 16)] = one16

    def _zrow(i, carry):
        for j in range(8):
            zrows[i, pl.ds(j * 16, 16)] = zero16
        return carry

    lax.fori_loop(0, zrows.shape[0], _zrow, 0)

    ZR = zrows.shape[0]

    def _zacc(i, carry):
        pltpu.async_copy(zrows, acc.at[pl.ds(row0 + i * ZR, ZR)], dsem)
        return carry

    lax.fori_loop(0, ROWS_PER_TILE // ZR, _zacc, 0)

    def _zdeg(i, carry):
        pltpu.async_copy(zrows.at[0], deg.at[pl.ds(row0 + i * 128, 128)],
                         dsem)
        return carry

    lax.fori_loop(0, ROWS_PER_TILE // 128, _zdeg, 0)

    def _zdrain_a(i, carry):
        pltpu.make_async_copy(zrows, acc.at[pl.ds(row0, ZR)], dsem).wait()
        return carry

    lax.fori_loop(0, ROWS_PER_TILE // ZR, _zdrain_a, 0)

    def _zdrain_d(i, carry):
        pltpu.make_async_copy(zrows.at[0], deg.at[pl.ds(row0, 128)],
                              dsem).wait()
        return carry

    lax.fori_loop(0, ROWS_PER_TILE // 128, _zdrain_d, 0)
    plsc.subcore_barrier()

    stages = []
    for rel, (x_hbm, src_hbm, dst_hbm) in enumerate(
        ((xu, sF, dF), (xu, sR, dR), (xi, sS, dS))
    ):
        for h in range(10):
            stages.append((rel, h, x_hbm, src_hbm, dst_hbm))

    pltpu.async_copy(sF.at[wid, pl.ds(0, QTR)], idxs0, isem)
    pltpu.async_copy(dF.at[wid, pl.ds(0, QTR)], idxd0, isem)

    for t, (rel, h, x_hbm, src_hbm, dst_hbm) in enumerate(stages):
        idxs, idxd = idxsets[t % 2]
        pltpu.make_async_copy(src_hbm.at[wid, pl.ds(h * QTR, QTR)], idxs,
                              isem).wait()
        pltpu.make_async_copy(dst_hbm.at[wid, pl.ds(h * QTR, QTR)], idxd,
                              isem).wait()
        if t + 1 < len(stages):
            _, nh, _, nsrc, ndst = stages[t + 1]
            nxs, nxd = idxsets[(t + 1) % 2]
            pltpu.async_copy(nsrc.at[wid, pl.ds(nh * QTR, QTR)], nxs, isem)
            pltpu.async_copy(ndst.at[wid, pl.ds(nh * QTR, QTR)], nxd, isem)

        def _gather(i, b, x_hbm=x_hbm, idxs=idxs):
            pltpu.async_copy(x_hbm.at[idxs.at[i]], rows[b], gsems[b])

        def _gwait(i, b, x_hbm=x_hbm, idxs=idxs):
            pltpu.make_async_copy(x_hbm.at[idxs.at[i]], rows[b],
                                  gsems[b]).wait()

        def _swait(b, idxd=idxd):
            pltpu.make_async_copy(rows[b], acc.at[idxd.at[0]],
                                  ssems[b]).wait()

        def _process(i, b, idxd=idxd):
            _gwait(i, b)
            pltpu.async_copy(ones, deg.at[idxd.at[i]], dsem, add=True)
            pltpu.async_copy(rows[b], acc.at[idxd.at[i]], ssems[b], add=True)

        _gather(0, 0)
        _gather(1, 1)
        _process(0, 0)
        _gather(2, 2)
        _process(1, 1)
        _gather(3, 3)

        def _quad(k, carry, _process=_process, _swait=_swait, _gather=_gather):
            i0 = 2 + 4 * k
            for j in range(4):
                i = i0 + j
                b = (2 + j) % 4
                _process(i, b)
                bn = (b + 2) % 4
                _swait(bn)
                _gather(i + 2, bn)
            return carry

        lax.fori_loop(0, (QTR - 4) // 4, _quad, 0)
        _process(QTR - 2, (QTR - 2) % 4)
        _process(QTR - 1, (QTR - 1) % 4)
        for b in range(4):
            _swait(b)

        def _ddrain(i, carry, idxd=idxd):
            pltpu.make_async_copy(ones, deg.at[idxd.at[0]], dsem).wait()
            return carry

        lax.fori_loop(0, QTR, _ddrain, 0)

        if h == 9:
            plsc.subcore_barrier()
            pltpu.sync_copy(acc.at[pl.ds(row0, ROWS_PER_TILE)],
                            accp.at[rel, c, pl.ds(row0, ROWS_PER_TILE)])
            pltpu.sync_copy(deg.at[pl.ds(row0, ROWS_PER_TILE)],
                            degp.at[rel, c, pl.ds(row0, ROWS_PER_TILE)])
            plsc.subcore_barrier()


_sc_scatter = functools.partial(
    pl.kernel,
    out_type=(
        jax.ShapeDtypeStruct((3, NC, PN, D), jnp.float32),
        jax.ShapeDtypeStruct((3, NC, PN), jnp.float32),
    ),
    mesh=plsc.VectorSubcoreMesh(core_axis_name="c", subcore_axis_name="s"),
    scratch_types=[
        pltpu.VMEM((QTR, CH), jnp.int32),
        pltpu.VMEM((QTR, CH), jnp.int32),
        pltpu.VMEM((QTR, CH), jnp.int32),
        pltpu.VMEM((QTR, CH), jnp.int32),
        pltpu.VMEM((CH, D), jnp.float32),
        pltpu.VMEM((CH, D), jnp.float32),
        pltpu.VMEM((CH, D), jnp.float32),
        pltpu.VMEM((CH, D), jnp.float32),
        pltpu.VMEM((CH,), jnp.float32),
        pltpu.VMEM((4, D), jnp.float32),
        pltpu.SemaphoreType.DMA,
        pltpu.SemaphoreType.DMA,
        pltpu.SemaphoreType.DMA,
        pltpu.SemaphoreType.DMA,
        pltpu.SemaphoreType.DMA,
        pltpu.SemaphoreType.DMA,
        pltpu.SemaphoreType.DMA,
        pltpu.SemaphoreType.DMA,
        pltpu.SemaphoreType.DMA,
        pltpu.SemaphoreType.DMA,
        pltpu.VMEM_SHARED((PN, D), jnp.float32),
        pltpu.VMEM_SHARED((PN,), jnp.float32),
    ],
)(_sc_body)


def _tc_body(accp, degp, wf, wr, ws, ou, oi):
    def norm(r):
        a = accp[r, 0] + accp[r, 1]
        dg = degp[r, 0] + degp[r, 1]
        if r > 0:
            a = a - (accp[r - 1, 0] + accp[r - 1, 1])
            dg = dg - (degp[r - 1, 0] + degp[r - 1, 1])
        return a / jnp.maximum(dg, 1.0)[:, None]

    ou[...] = jnp.dot(norm(0), wf[...], preferred_element_type=jnp.float32)
    oi[...] = (jnp.dot(norm(1), wr[...], preferred_element_type=jnp.float32)
               + jnp.dot(norm(2), ws[...], preferred_element_type=jnp.float32))


_TB = 1280

_tc_combine = pl.pallas_call(
    _tc_body,
    grid=(PN // _TB,),
    in_specs=[
        pl.BlockSpec((3, NC, _TB, D), lambda i: (0, 0, i, 0)),
        pl.BlockSpec((3, NC, _TB), lambda i: (0, 0, i)),
        pl.BlockSpec((D, D), lambda i: (0, 0)),
        pl.BlockSpec((D, D), lambda i: (0, 0)),
        pl.BlockSpec((D, D), lambda i: (0, 0)),
    ],
    out_specs=[
        pl.BlockSpec((_TB, D), lambda i: (i, 0)),
        pl.BlockSpec((_TB, D), lambda i: (i, 0)),
    ],
    out_shape=[
        jax.ShapeDtypeStruct((PN, D), jnp.float32),
        jax.ShapeDtypeStruct((PN, D), jnp.float32),
    ],
)


def _prep_edges(e):
    pad = jnp.arange(EPW - E // NW, dtype=jnp.int32) % NPAD
    pad = jnp.broadcast_to(pad, (NW, EPW - E // NW))

    def one(v, base):
        v = v.reshape(NW, E // NW)
        v = jnp.concatenate([v, pad + base], axis=1)
        return v.reshape(NW, STEPS, CH)

    return one(e[0], 0), one(e[1], N)


def kernel(x_user, x_item, edge_follows, edge_rates, edge_similar,
           W_follows, W_rates, W_similar):
    sF, dF = _prep_edges(edge_follows)
    sR, dR = _prep_edges(edge_rates)
    sS, dS = _prep_edges(edge_similar)

    accp, degp = _sc_scatter(x_user, x_item, sF, dF, sR, dR, sS, dS)
    ou, oi = _tc_combine(accp, degp, W_follows, W_rates, W_similar)
    return ou[:N], oi[:N]
